# baseline (speedup 1.0000x reference)
import functools

import jax
import jax.numpy as jnp
from jax import lax
from jax.experimental import pallas as pl
from jax.experimental.pallas import tpu as pltpu
from jax.experimental.pallas import tpu_sc as plsc

B = 16384
D = 32
NC, NS = 2, 16
NW = NC * NS
BPW = B // NW
CH = 128
NCHUNK = BPW // CH

USER_V, ORG_V = 1000000, 100000
FIELD_V, ROLE_V, DAY_V, HOUR_V = 1000, 1000, 32, 24

_mesh = plsc.VectorSubcoreMesh(
    core_axis_name="c", subcore_axis_name="s", num_cores=NC, num_subcores=NS
)


@functools.partial(
    pl.kernel,
    out_type=jax.ShapeDtypeStruct((B, 7 * D), jnp.float32),
    mesh=_mesh,
    scratch_types=[
        pltpu.VMEM((7, BPW), jnp.int32),
        pltpu.VMEM((2, CH), jnp.int32),
        pltpu.VMEM((2, CH, 128), jnp.float32),
        pltpu.VMEM((FIELD_V * D // 128, 128), jnp.float32),
        pltpu.VMEM((ROLE_V * D // 128, 128), jnp.float32),
        pltpu.VMEM((DAY_V * D // 128, 128), jnp.float32),
        pltpu.VMEM((HOUR_V * D // 128, 128), jnp.float32),
        pltpu.VMEM((2, CH, D), jnp.float32),
        pltpu.SemaphoreType.DMA,
        pltpu.SemaphoreType.DMA,
        pltpu.SemaphoreType.DMA,
        pltpu.SemaphoreType.DMA,
    ],
    compiler_params=pltpu.CompilerParams(use_tc_tiling_on_sc=False),
)
def _usermodel(u, o, f0, f1, r, d, t, Wu, Wo, wf, wr, wd, wh,
               out, idx_v, qid_v, quad_v, wf_v, wr_v, wd_v, wh_v, ext_v,
               sem_i, sem_g, sem_t, sem_o):
    wid = lax.axis_index("s") * NC + lax.axis_index("c")
    base = wid * BPW
    idx_hbm = (u, o, f0, f1, r, d, t)

    icps = [
        pltpu.async_copy(idx_hbm[i].at[pl.ds(base, BPW)], idx_v.at[i], sem_i)
        for i in range(7)
    ]
    tcps = [
        pltpu.async_copy(src, dst, sem_t)
        for src, dst in ((wf, wf_v), (wr, wr_v), (wd, wd_v), (wh, wh_v))
    ]
    for c in icps:
        c.wait()

    big_items = [(feat, tab, h)
                 for h in range(NCHUNK) for feat, tab in ((0, Wu), (1, Wo))]

    def fill_qids(feat, h, slot):
        def qbody(g, carry, feat=feat, h=h, slot=slot):
            v = idx_v[feat, pl.ds(h * CH + g * 16, 16)]
            qid_v[slot, pl.ds(g * 16, 16)] = lax.shift_right_logical(v, 2)
            return carry
        lax.fori_loop(0, CH // 16, qbody, 0)

    def extract_big(feat, h, slot):
        def ebody(g, carry, feat=feat, h=h, slot=slot):
            idx16 = idx_v[feat, pl.ds(h * CH + g * 16, 16)]
            sub16 = (idx16 & 3) * D
            for k in range(16):
                off = sub16[k]
                rr = g * 16 + k
                ext_v[slot, rr, pl.ds(0, 16)] = quad_v[slot, rr,
                                                       pl.ds(off, 16)]
                ext_v[slot, rr, pl.ds(16, 16)] = quad_v[slot, rr,
                                                        pl.ds(off + 16, 16)]
            return carry
        lax.fori_loop(0, CH // 16, ebody, 0)

    gcp = [None, None]
    ocp = [None, None]
    meta = [None, None]
    for n, (feat, tab, h) in enumerate(big_items):
        slot = n % 2
        if gcp[slot] is not None:
            gcp[slot].wait()
            pfeat, ph = meta[slot]
            extract_big(pfeat, ph, slot)
            if ocp[slot] is not None:
                ocp[slot].wait()
            ocp[slot] = pltpu.async_copy(
                ext_v.at[slot],
                out.at[pl.ds(base + ph * CH, CH),
                       pl.ds(pfeat * D, D)], sem_o)
        fill_qids(feat, h, slot)
        gcp[slot] = pltpu.async_copy(tab.at[qid_v.at[slot]], quad_v.at[slot],
                                     sem_g)
        meta[slot] = (feat, h)
    for slot in (0, 1):
        if gcp[slot] is not None:
            gcp[slot].wait()
            pfeat, ph = meta[slot]
            extract_big(pfeat, ph, slot)
            if ocp[slot] is not None:
                ocp[slot].wait()
            ocp[slot] = pltpu.async_copy(
                ext_v.at[slot],
                out.at[pl.ds(base + ph * CH, CH),
                       pl.ds(pfeat * D, D)], sem_o)

    for c in tcps:
        c.wait()
    smalls = ((2, wf_v), (3, wf_v), (4, wr_v), (5, wd_v), (6, wh_v))
    for si, (feat, tab) in enumerate(smalls):
        for h in range(NCHUNK):
            slot = (si * NCHUNK + h) % 2
            if ocp[slot] is not None:
                ocp[slot].wait()
                ocp[slot] = None

            def sbody(g, carry, feat=feat, tab=tab, h=h, slot=slot):
                idx16 = idx_v[feat, pl.ds(h * CH + g * 16, 16)] * D
                for k in range(16):
                    s = idx16[k]
                    q = lax.shift_right_logical(s, 7)
                    cc = s & 127
                    rr = g * 16 + k
                    ext_v[slot, rr, pl.ds(0, 16)] = tab[q, pl.ds(cc, 16)]
                    ext_v[slot, rr, pl.ds(16, 16)] = tab[q,
                                                         pl.ds(cc + 16, 16)]
                return carry

            lax.fori_loop(0, CH // 16, sbody, 0)
            ocp[slot] = pltpu.async_copy(
                ext_v.at[slot],
                out.at[pl.ds(base + h * CH, CH),
                       pl.ds(feat * D, D)], sem_o)
    for slot in (0, 1):
        if ocp[slot] is not None:
            ocp[slot].wait()


def kernel(user_id, organization, interested_fields_0, interested_fields_1,
           role, date, time, W_user, W_org, W_field, W_role, W_day, W_hour):
    return _usermodel(
        user_id, organization, interested_fields_0, interested_fields_1,
        role, date, time,
        W_user.reshape(USER_V * D // 128, 128),
        W_org.reshape(ORG_V * D // 128, 128),
        W_field.reshape(FIELD_V * D // 128, 128),
        W_role.reshape(ROLE_V * D // 128, 128),
        W_day.reshape(DAY_V * D // 128, 128),
        W_hour.reshape(HOUR_V * D // 128, 128))
